# baseline (speedup 1.0000x reference)
import jax
import jax.numpy as jnp
from jax.experimental import pallas as pl
from jax.experimental.pallas import tpu as pltpu

N_CHANNELS = 4096
N_BINS = 8192
BLOCK_R = 256
BLOCK_C = 4096
N_CSTEPS = N_BINS // BLOCK_C


def _mv_kernel(params_ref, elow_ref, ehigh_ref, tm_ref, out_ref):
    j = pl.program_id(1)
    norm = params_ref[0]
    alpha = params_ref[1]
    e_low = elow_ref[...]
    e_high = ehigh_ref[...]
    e_mid = 0.5 * (e_low + e_high)
    de = e_high - e_low
    flux = norm * jnp.exp(-alpha * jnp.log(e_mid)) * de
    acc = jax.lax.dot_general(
        tm_ref[...], flux,
        dimension_numbers=(((1,), (1,)), ((), ())),
        preferred_element_type=jnp.float32,
    )

    @pl.when(j == 0)
    def _():
        out_ref[...] = acc

    @pl.when(j == N_CSTEPS - 1)
    def _():
        prev = acc if N_CSTEPS == 1 else out_ref[...] + acc
        out_ref[...] = jnp.maximum(prev, 1e-6)

    if N_CSTEPS > 2:
        @pl.when((j > 0) & (j < N_CSTEPS - 1))
        def _():
            out_ref[...] += acc


def kernel(parameters, transfer_matrix, e_low, e_high):
    e_low2 = e_low.reshape(1, N_BINS)
    e_high2 = e_high.reshape(1, N_BINS)
    out = pl.pallas_call(
        _mv_kernel,
        grid=(N_CHANNELS // BLOCK_R, N_CSTEPS),
        in_specs=[
            pl.BlockSpec(memory_space=pltpu.SMEM),
            pl.BlockSpec((1, BLOCK_C), lambda i, j: (0, j)),
            pl.BlockSpec((1, BLOCK_C), lambda i, j: (0, j)),
            pl.BlockSpec((BLOCK_R, BLOCK_C), lambda i, j: (i, j)),
        ],
        out_specs=pl.BlockSpec((BLOCK_R, 1), lambda i, j: (i, 0)),
        out_shape=jax.ShapeDtypeStruct((N_CHANNELS, 1), jnp.float32),
    )(parameters, e_low2, e_high2, transfer_matrix)
    return out.reshape(N_CHANNELS)


# speedup vs baseline: 1.1347x; 1.1347x over previous
import jax
import jax.numpy as jnp
from jax.experimental import pallas as pl
from jax.experimental.pallas import tpu as pltpu

N_CHANNELS = 4096
N_BINS = 8192
BLOCK_R = 256


def _mv_kernel(params_ref, elow_ref, ehigh_ref, tm_ref, out_ref):
    norm = params_ref[0]
    alpha = params_ref[1]
    e_low = elow_ref[...]
    e_high = ehigh_ref[...]
    e_mid = 0.5 * (e_low + e_high)
    de = e_high - e_low
    flux = norm * jnp.exp(-alpha * jnp.log(e_mid)) * de
    acc = jax.lax.dot_general(
        tm_ref[...], flux,
        dimension_numbers=(((1,), (1,)), ((), ())),
        preferred_element_type=jnp.float32,
    )
    out_ref[...] = jnp.maximum(acc, 1e-6)


def kernel(parameters, transfer_matrix, e_low, e_high):
    e_low2 = e_low.reshape(1, N_BINS)
    e_high2 = e_high.reshape(1, N_BINS)
    out = pl.pallas_call(
        _mv_kernel,
        grid=(N_CHANNELS // BLOCK_R,),
        in_specs=[
            pl.BlockSpec(memory_space=pltpu.SMEM),
            pl.BlockSpec((1, N_BINS), lambda i: (0, 0)),
            pl.BlockSpec((1, N_BINS), lambda i: (0, 0)),
            pl.BlockSpec((BLOCK_R, N_BINS), lambda i: (i, 0)),
        ],
        out_specs=pl.BlockSpec((BLOCK_R, 1), lambda i: (i, 0)),
        out_shape=jax.ShapeDtypeStruct((N_CHANNELS, 1), jnp.float32),
    )(parameters, e_low2, e_high2, transfer_matrix)
    return out.reshape(N_CHANNELS)
